# baseline (speedup 1.0000x reference)
import jax
import jax.numpy as jnp
from jax import lax
from jax.experimental import pallas as pl
from jax.experimental.pallas import tpu as pltpu

_BN = 256
_BK = 1024


def _linear_kernel(x_ref, w_ref, b_ref, o_ref, xs_ref):
    n = pl.program_id(0)
    k = pl.program_id(1)

    @pl.when(n == 0)
    def _():
        xs_ref[:, pl.ds(k * _BK, _BK)] = x_ref[...]

    part = lax.dot_general(
        xs_ref[:, pl.ds(k * _BK, _BK)], w_ref[...],
        dimension_numbers=(((1,), (1,)), ((), ())),
        preferred_element_type=jnp.float32,
        precision=lax.Precision.DEFAULT,
    )

    @pl.when(k == 0)
    def _():
        o_ref[...] = part + b_ref[...]

    @pl.when(k != 0)
    def _():
        o_ref[...] += part


def kernel(x, weight, bias):
    batch, in_f = x.shape
    out_f = weight.shape[0]
    nb, kb = out_f // _BN, in_f // _BK
    brow = bias.reshape(1, out_f)
    return pl.pallas_call(
        _linear_kernel,
        grid=(nb, kb),
        in_specs=[
            pl.BlockSpec((batch, _BK),
                         lambda n, k: (0, jnp.where(n == 0, k, kb - 1))),
            pl.BlockSpec((_BN, _BK), lambda n, k: (n, k)),
            pl.BlockSpec((1, _BN), lambda n, k: (0, n)),
        ],
        out_specs=pl.BlockSpec((batch, _BN), lambda n, k: (0, n)),
        out_shape=jax.ShapeDtypeStruct((batch, out_f), jnp.float32),
        scratch_shapes=[pltpu.VMEM((batch, in_f), jnp.float32)],
        compiler_params=pltpu.CompilerParams(
            dimension_semantics=("arbitrary", "arbitrary"),
        ),
    )(x, weight, brow)


# speedup vs baseline: 1.6683x; 1.6683x over previous
import jax
import jax.numpy as jnp
from jax import lax
from jax.experimental import pallas as pl
from jax.experimental.pallas import tpu as pltpu

_BN = 512


def _linear_kernel(x_ref, w_ref, b_ref, o_ref):
    acc = lax.dot_general(
        x_ref[...], w_ref[...],
        dimension_numbers=(((1,), (1,)), ((), ())),
        preferred_element_type=jnp.float32,
        precision=lax.Precision.DEFAULT,
    )
    o_ref[...] = acc + b_ref[...]


def kernel(x, weight, bias):
    batch, in_f = x.shape
    out_f = weight.shape[0]
    brow = bias.reshape(1, out_f)
    return pl.pallas_call(
        _linear_kernel,
        grid=(out_f // _BN,),
        in_specs=[
            pl.BlockSpec((batch, in_f), lambda n: (0, 0)),
            pl.BlockSpec((_BN, in_f), lambda n: (n, 0)),
            pl.BlockSpec((1, _BN), lambda n: (0, n)),
        ],
        out_specs=pl.BlockSpec((batch, _BN), lambda n: (0, n)),
        out_shape=jax.ShapeDtypeStruct((batch, out_f), jnp.float32),
        compiler_params=pltpu.CompilerParams(
            dimension_semantics=("arbitrary",),
        ),
    )(x, weight, brow)
